# baseline (speedup 1.0000x reference)
import functools

import jax
import jax.numpy as jnp
from jax import lax
from jax.experimental import pallas as pl
from jax.experimental.pallas import tpu as pltpu
from jax.experimental.pallas import tpu_sc as plsc

NC = 2
NS = 16
L = 16
NW = NC * NS


def _emb_kernel(B, T, D, P, idx_hbm, wpe_hbm, wte_hbm, out_hbm,
                idx_v, wpe_v, rows_v, sem):
    wid = lax.axis_index("s") * NC + lax.axis_index("c")
    pos_base = wid * P

    pltpu.sync_copy(wpe_hbm.at[pl.ds(pos_base, P)], wpe_v)

    vecs_per_row = D // L

    def add_row(r, _):
        def add_vec(j, __):
            sl = pl.ds(j * L, L)
            rows_v[r, sl] = rows_v[r, sl] + wpe_v[r, sl]
            return __
        return lax.fori_loop(0, vecs_per_row, add_vec, _)

    for b in range(B):
        row_base = b * T + pos_base
        pltpu.sync_copy(idx_hbm.at[pl.ds(row_base, P)], idx_v)
        pltpu.async_copy(wte_hbm.at[idx_v], rows_v, sem).wait()
        lax.fori_loop(0, P, add_row, 0)
        pltpu.sync_copy(rows_v, out_hbm.at[pl.ds(row_base, P)])


def kernel(idx, wte, wpe):
    B, T = idx.shape
    V, D = wte.shape
    P = T // NW

    mesh = plsc.VectorSubcoreMesh(core_axis_name="c", subcore_axis_name="s")
    body = functools.partial(_emb_kernel, B, T, D, P)
    out = pl.kernel(
        body,
        out_type=jax.ShapeDtypeStruct((B * T, D), jnp.float32),
        mesh=mesh,
        scratch_types=[
            pltpu.VMEM((P,), jnp.int32),
            pltpu.VMEM((P, D), jnp.float32),
            pltpu.VMEM((P, D), jnp.float32),
            pltpu.SemaphoreType.DMA,
        ],
    )(idx.reshape(B * T), wte, wpe)
    return out.reshape(B, T, D)


# speedup vs baseline: 1.5752x; 1.5752x over previous
import functools

import jax
import jax.numpy as jnp
from jax import lax
from jax.experimental import pallas as pl
from jax.experimental.pallas import tpu as pltpu
from jax.experimental.pallas import tpu_sc as plsc

NC = 2
NS = 16
L = 16
NW = NC * NS
C = 32


def _emb_kernel(B, T, D, P, idx_hbm, wpe_hbm, wte_hbm, out_hbm,
                idx_v, wpe_v, rows_v, gsems, ssems):
    wid = lax.axis_index("s") * NC + lax.axis_index("c")
    pos_base = wid * P
    n_chunks = P // C
    n_jobs = n_chunks * B
    vecs_per_row = D // L

    pltpu.sync_copy(wpe_hbm.at[pl.ds(pos_base, P)], wpe_v)
    for j in range(n_jobs):
        pc, b = divmod(j, B)
        row_base = b * T + pos_base + pc * C
        pltpu.sync_copy(idx_hbm.at[pl.ds(row_base, C)], idx_v.at[j])

    def add_chunk(pc, buf):
        def add_row(r, _):
            for v in range(vecs_per_row):
                sl = pl.ds(v * L, L)
                buf[r, sl] = buf[r, sl] + wpe_v[pc * C + r, sl]
            return _
        lax.fori_loop(0, C, add_row, 0)

    gathers = [None] * n_jobs
    stores = [None] * n_jobs

    def start_gather(j):
        gathers[j] = pltpu.async_copy(
            wte_hbm.at[idx_v.at[j]], rows_v.at[j % 2], gsems.at[j % 2])

    start_gather(0)
    for j in range(n_jobs):
        pc, b = divmod(j, B)
        gathers[j].wait()
        if j > 0:
            stores[j - 1].wait()
        if j + 1 < n_jobs:
            start_gather(j + 1)
        add_chunk(pc, rows_v.at[j % 2])
        row_base = b * T + pos_base + pc * C
        stores[j] = pltpu.async_copy(
            rows_v.at[j % 2], out_hbm.at[pl.ds(row_base, C)], ssems.at[j % 2])
    stores[n_jobs - 1].wait()


def kernel(idx, wte, wpe):
    B, T = idx.shape
    V, D = wte.shape
    P = T // NW
    n_jobs = (P // C) * B

    mesh = plsc.VectorSubcoreMesh(core_axis_name="c", subcore_axis_name="s")
    body = functools.partial(_emb_kernel, B, T, D, P)
    out = pl.kernel(
        body,
        out_type=jax.ShapeDtypeStruct((B * T, D), jnp.float32),
        mesh=mesh,
        scratch_types=[
            pltpu.VMEM((n_jobs, C), jnp.int32),
            pltpu.VMEM((P, D), jnp.float32),
            pltpu.VMEM((2, C, D), jnp.float32),
            pltpu.SemaphoreType.DMA((2,)),
            pltpu.SemaphoreType.DMA((2,)),
        ],
    )(idx.reshape(B * T), wte, wpe)
    return out.reshape(B, T, D)
